# Initial kernel scaffold; baseline (speedup 1.0000x reference)
import jax
import jax.numpy as jnp
from jax.experimental import pallas as pl


def kernel(in_feat, edge_index, W1, b1, W2, b2):
    raise NotImplementedError("write your pallas kernel here")


# speedup vs baseline: 2.3297x; 2.3297x over previous
import functools

import jax
import jax.numpy as jnp
from jax import lax
from jax.experimental import pallas as pl
from jax.experimental.pallas import tpu as pltpu
from jax.experimental.pallas import tpu_sc as plsc

N = 10000
E = 160000
D_IN = 256
D_H = 256
D_OUT = 64

NS = 16
NC = 2
K = 40
EPT = E // NS
CCH = EPT // K
RCH = 624
RTAIL = N - NS * RCH

BM = 400
GRID = N // BM

_mesh = plsc.VectorSubcoreMesh(core_axis_name="c", subcore_axis_name="s")


@functools.partial(
    pl.kernel,
    out_type=jax.ShapeDtypeStruct((NC, N, 16), jnp.float32),
    mesh=_mesh,
    compiler_params=pltpu.CompilerParams(use_tc_tiling_on_sc=False),
    scratch_types=[
        pltpu.VMEM((CCH, K), jnp.int32),
        pltpu.VMEM((K, 16), jnp.float32),
        pltpu.VMEM_SHARED((N, 16), jnp.float32),
    ],
)
def _deg_kernel(e4, zrows, ones_rows, deg_out, idx, ones_v, deg_sp):
    c = lax.axis_index("c")
    s = lax.axis_index("s")
    pltpu.sync_copy(zrows.at[pl.ds(0, RCH)], deg_sp.at[pl.ds(s * RCH, RCH)])

    @pl.when(s == NS - 1)
    def _():
        pltpu.sync_copy(zrows.at[pl.ds(0, RTAIL)],
                        deg_sp.at[pl.ds(NS * RCH, RTAIL)])

    pltpu.sync_copy(e4.at[c, s], idx)
    pltpu.sync_copy(ones_rows, ones_v)
    plsc.subcore_barrier()

    def step(j, carry):
        pltpu.sync_copy(ones_v, deg_sp.at[idx.at[j]], add=True)
        return carry

    lax.fori_loop(0, CCH, step, 0)
    plsc.subcore_barrier()
    pltpu.sync_copy(deg_sp.at[pl.ds(s * RCH, RCH)],
                    deg_out.at[c, pl.ds(s * RCH, RCH)])

    @pl.when(s == NS - 1)
    def _():
        pltpu.sync_copy(deg_sp.at[pl.ds(NS * RCH, RTAIL)],
                        deg_out.at[c, pl.ds(NS * RCH, RTAIL)])


def _make_agg_kernel(dq):
    @functools.partial(
        pl.kernel,
        out_type=[jax.ShapeDtypeStruct((N, dq), jnp.float32)] * 4,
        mesh=_mesh,
        compiler_params=pltpu.CompilerParams(use_tc_tiling_on_sc=False),
        scratch_types=[
            pltpu.VMEM((CCH, K), jnp.int32),
            pltpu.VMEM((CCH, K), jnp.int32),
            pltpu.VMEM((K, dq), jnp.float32),
            pltpu.VMEM_SHARED((N, dq), jnp.float32),
            pltpu.SemaphoreType.DMA,
        ],
    )
    def agg_kernel(hw0, hw1, hw2, hw3, src4, dst4, zrows,
                   out0, out1, out2, out3, idx_s, idx_d, gbuf, acc, sem):
        c = lax.axis_index("c")
        s = lax.axis_index("s")
        pltpu.sync_copy(src4.at[s], idx_s)
        pltpu.sync_copy(dst4.at[s], idx_d)

        def run(hw, out):
            pltpu.sync_copy(zrows.at[pl.ds(0, RCH)],
                            acc.at[pl.ds(s * RCH, RCH)])

            @pl.when(s == NS - 1)
            def _():
                pltpu.sync_copy(zrows.at[pl.ds(0, RTAIL)],
                                acc.at[pl.ds(NS * RCH, RTAIL)])

            plsc.subcore_barrier()

            def step(j, carry):
                pltpu.async_copy(hw.at[idx_s.at[j]], gbuf, sem).wait()
                pltpu.sync_copy(gbuf, acc.at[idx_d.at[j]], add=True)
                return carry

            lax.fori_loop(0, CCH, step, 0)
            plsc.subcore_barrier()
            pltpu.sync_copy(acc.at[pl.ds(s * RCH, RCH)],
                            out.at[pl.ds(s * RCH, RCH)])

            @pl.when(s == NS - 1)
            def _():
                pltpu.sync_copy(acc.at[pl.ds(NS * RCH, RTAIL)],
                                out.at[pl.ds(NS * RCH, RTAIL)])

            plsc.subcore_barrier()

        @pl.when(c == 0)
        def _():
            run(hw0, out0)
            run(hw1, out1)

        @pl.when(c == 1)
        def _():
            run(hw2, out2)
            run(hw3, out3)

    return agg_kernel


_agg_h = _make_agg_kernel(D_H // 4)
_agg_o = _make_agg_kernel(D_OUT // 4)


def _tc1_body(x_ref, deg_ref, w_ref, o0, o1, o2, o3):
    n_out = lax.rsqrt(jnp.maximum(deg_ref[0, :, 0:1], 1.0))
    y = jnp.dot(x_ref[...] * n_out, w_ref[...],
                preferred_element_type=jnp.float32)
    dq = D_H // 4
    for q, o in enumerate((o0, o1, o2, o3)):
        o[...] = y[:, q * dq:(q + 1) * dq]


def _tc1(x, deg16, w1):
    return pl.pallas_call(
        _tc1_body,
        grid=(GRID,),
        in_specs=[
            pl.BlockSpec((BM, D_IN), lambda i: (i, 0)),
            pl.BlockSpec((1, BM, 16), lambda i: (0, i, 0)),
            pl.BlockSpec((D_IN, D_H), lambda i: (0, 0)),
        ],
        out_specs=[pl.BlockSpec((BM, D_H // 4), lambda i: (i, 0))] * 4,
        out_shape=[jax.ShapeDtypeStruct((N, D_H // 4), jnp.float32)] * 4,
    )(x, deg16, w1)


def _tc2_body(a0, a1, a2, a3, deg_ref, b1_ref, w_ref, o0, o1, o2, o3):
    n_out = lax.rsqrt(jnp.maximum(deg_ref[0, :, 0:1], 1.0))
    n_in = lax.rsqrt(jnp.maximum(deg_ref[1, :, 0:1], 1.0))
    agg = jnp.concatenate([a0[...], a1[...], a2[...], a3[...]], axis=1)
    h = jax.nn.relu(agg * n_in + b1_ref[...]) * n_out
    y = jnp.dot(h, w_ref[...], preferred_element_type=jnp.float32)
    dq = D_OUT // 4
    for q, o in enumerate((o0, o1, o2, o3)):
        o[...] = y[:, q * dq:(q + 1) * dq]


def _tc2(aggs, deg16, b1, w2):
    return pl.pallas_call(
        _tc2_body,
        grid=(GRID,),
        in_specs=[pl.BlockSpec((BM, D_H // 4), lambda i: (i, 0))] * 4 + [
            pl.BlockSpec((2, BM, 16), lambda i: (0, i, 0)),
            pl.BlockSpec((1, D_H), lambda i: (0, 0)),
            pl.BlockSpec((D_H, D_OUT), lambda i: (0, 0)),
        ],
        out_specs=[pl.BlockSpec((BM, D_OUT // 4), lambda i: (i, 0))] * 4,
        out_shape=[jax.ShapeDtypeStruct((N, D_OUT // 4), jnp.float32)] * 4,
    )(*aggs, deg16, b1, w2)


def _tc3_body(a0, a1, a2, a3, deg_ref, b2_ref, o_ref):
    n_in = lax.rsqrt(jnp.maximum(deg_ref[0, :, 0:1], 1.0))
    agg = jnp.concatenate([a0[...], a1[...], a2[...], a3[...]], axis=1)
    o_ref[...] = agg * n_in + b2_ref[...]


def _tc3(aggs, deg16, b2):
    return pl.pallas_call(
        _tc3_body,
        grid=(GRID,),
        in_specs=[pl.BlockSpec((BM, D_OUT // 4), lambda i: (i, 0))] * 4 + [
            pl.BlockSpec((1, BM, 16), lambda i: (1, i, 0)),
            pl.BlockSpec((1, D_OUT), lambda i: (0, 0)),
        ],
        out_specs=pl.BlockSpec((BM, D_OUT), lambda i: (i, 0)),
        out_shape=jax.ShapeDtypeStruct((N, D_OUT), jnp.float32),
    )(*aggs, deg16, b2)


def kernel(in_feat, edge_index, W1, b1, W2, b2):
    e4 = edge_index.reshape(NC, NS, CCH, K)
    src4 = e4[0]
    dst4 = e4[1]
    z16 = jnp.zeros((RCH, 16), jnp.float32)
    zh = jnp.zeros((RCH, D_H // 4), jnp.float32)
    zo = jnp.zeros((RCH, D_OUT // 4), jnp.float32)
    ones_rows = jnp.zeros((K, 16), jnp.float32).at[:, 0].set(1.0)

    deg16 = _deg_kernel(e4, z16, ones_rows)
    hw = _tc1(in_feat, deg16, W1)
    a1 = _agg_h(*hw, src4, dst4, zh)
    g = _tc2(a1, deg16, b1.reshape(1, D_H), W2)
    a2 = _agg_o(*g, src4, dst4, zo)
    return _tc3(a2, deg16, b2.reshape(1, D_OUT))


# speedup vs baseline: 7.4084x; 3.1800x over previous
import functools

import jax
import jax.numpy as jnp
from jax import lax
from jax.experimental import pallas as pl
from jax.experimental.pallas import tpu as pltpu
from jax.experimental.pallas import tpu_sc as plsc

N = 10000
E = 160000
D_IN = 256
D_H = 256
D_OUT = 64

NS = 16
NC = 2
NB = 5

KD = 40
CCD = (E // NS) // KD

KH = 80
CCH = (E // NS) // KH

KO = 40
CCO = (E // NC // NS) // KO

RCH = 624
RTAIL = N - NS * RCH

BM = 400
GRID = N // BM

_mesh = plsc.VectorSubcoreMesh(core_axis_name="c", subcore_axis_name="s")
_params = pltpu.CompilerParams(use_tc_tiling_on_sc=False)


def _zero_acc(zrows, acc, s):
    pltpu.sync_copy(zrows.at[pl.ds(0, RCH)], acc.at[pl.ds(s * RCH, RCH)])

    @pl.when(s == NS - 1)
    def _():
        pltpu.sync_copy(zrows.at[pl.ds(0, RTAIL)],
                        acc.at[pl.ds(NS * RCH, RTAIL)])


def _dump_acc(acc, out, s):
    pltpu.sync_copy(acc.at[pl.ds(s * RCH, RCH)], out.at[pl.ds(s * RCH, RCH)])

    @pl.when(s == NS - 1)
    def _():
        pltpu.sync_copy(acc.at[pl.ds(NS * RCH, RTAIL)],
                        out.at[pl.ds(NS * RCH, RTAIL)])


def _edge_loop(hw, acc, idx_s, idx_d, gbufs, gsems, ssems, cch):
    nb = len(gbufs)
    t_outer = cch // nb
    for b in range(nb - 1):
        pltpu.async_copy(hw.at[idx_s.at[b]], gbufs[b], gsems[b])

    def outer(t, carry):
        for b in range(nb):
            jj = t * nb + b
            bprev = (b - 1) % nb

            def wait_scatter(bp=bprev, j=jj):
                pltpu.make_async_copy(gbufs[bp], acc.at[idx_d.at[j - 1]],
                                      ssems[bp]).wait()

            def issue_gather(bp=bprev, j=jj):
                pltpu.async_copy(hw.at[idx_s.at[j + nb - 1]], gbufs[bp],
                                 gsems[bp])

            if b == 0:
                pl.when(t > 0)(wait_scatter)
                issue_gather()
            else:
                wait_scatter()
                pl.when(t < t_outer - 1)(issue_gather)
            pltpu.make_async_copy(hw.at[idx_s.at[jj]], gbufs[b],
                                  gsems[b]).wait()
            pltpu.async_copy(gbufs[b], acc.at[idx_d.at[jj]], ssems[b],
                             add=True)
        return carry

    lax.fori_loop(0, t_outer, outer, 0)
    bl = (cch - 1) % nb
    pltpu.make_async_copy(gbufs[bl], acc.at[idx_d.at[cch - 1]],
                          ssems[bl]).wait()


@functools.partial(
    pl.kernel,
    out_type=jax.ShapeDtypeStruct((NC, N, 16), jnp.float32),
    mesh=_mesh,
    compiler_params=_params,
    scratch_types=[
        pltpu.VMEM((CCD, KD), jnp.int32),
        pltpu.VMEM((KD, 16), jnp.float32),
        pltpu.VMEM_SHARED((N, 16), jnp.float32),
    ],
)
def _deg_kernel(e4, zrows, ones_rows, deg_out, idx, ones_v, deg_sp):
    c = lax.axis_index("c")
    s = lax.axis_index("s")
    _zero_acc(zrows, deg_sp, s)
    pltpu.sync_copy(e4.at[c, s], idx)
    pltpu.sync_copy(ones_rows, ones_v)
    plsc.subcore_barrier()

    def step(j, carry):
        pltpu.sync_copy(ones_v, deg_sp.at[idx.at[j]], add=True)
        return carry

    lax.fori_loop(0, CCD, step, 0)
    plsc.subcore_barrier()
    _dump_acc(deg_sp, deg_out.at[c], s)


@functools.partial(
    pl.kernel,
    out_type=[jax.ShapeDtypeStruct((N, D_H // 4), jnp.float32)] * 4,
    mesh=_mesh,
    compiler_params=_params,
    scratch_types=[
        pltpu.VMEM((CCH, KH), jnp.int32),
        pltpu.VMEM((CCH, KH), jnp.int32),
        [pltpu.VMEM((KH, D_H // 4), jnp.float32)] * NB,
        pltpu.VMEM_SHARED((N, D_H // 4), jnp.float32),
        [pltpu.SemaphoreType.DMA] * NB,
        [pltpu.SemaphoreType.DMA] * NB,
    ],
)
def _agg_h(hw0, hw1, hw2, hw3, srcH, dstH, zrows,
           out0, out1, out2, out3, idx_s, idx_d, gbufs, acc, gsems, ssems):
    c = lax.axis_index("c")
    s = lax.axis_index("s")
    pltpu.sync_copy(srcH.at[s], idx_s)
    pltpu.sync_copy(dstH.at[s], idx_d)

    def run(hw, out):
        _zero_acc(zrows, acc, s)
        plsc.subcore_barrier()
        _edge_loop(hw, acc, idx_s, idx_d, gbufs, gsems, ssems, CCH)
        plsc.subcore_barrier()
        _dump_acc(acc, out, s)
        plsc.subcore_barrier()

    @pl.when(c == 0)
    def _():
        run(hw0, out0)
        run(hw1, out1)

    @pl.when(c == 1)
    def _():
        run(hw2, out2)
        run(hw3, out3)


@functools.partial(
    pl.kernel,
    out_type=jax.ShapeDtypeStruct((NC, N, D_OUT), jnp.float32),
    mesh=_mesh,
    compiler_params=_params,
    scratch_types=[
        pltpu.VMEM((CCO, KO), jnp.int32),
        pltpu.VMEM((CCO, KO), jnp.int32),
        [pltpu.VMEM((KO, D_OUT), jnp.float32)] * NB,
        pltpu.VMEM_SHARED((N, D_OUT), jnp.float32),
        [pltpu.SemaphoreType.DMA] * NB,
        [pltpu.SemaphoreType.DMA] * NB,
    ],
)
def _agg_o(hw, srcO, dstO, zrows, out,
           idx_s, idx_d, gbufs, acc, gsems, ssems):
    c = lax.axis_index("c")
    s = lax.axis_index("s")
    pltpu.sync_copy(srcO.at[c, s], idx_s)
    pltpu.sync_copy(dstO.at[c, s], idx_d)
    _zero_acc(zrows, acc, s)
    plsc.subcore_barrier()
    _edge_loop(hw, acc, idx_s, idx_d, gbufs, gsems, ssems, CCO)
    plsc.subcore_barrier()
    _dump_acc(acc, out.at[c], s)


def _tc1_body(x_ref, deg_ref, w_ref, o0, o1, o2, o3):
    n_out = lax.rsqrt(jnp.maximum(deg_ref[0, :, 0:1], 1.0))
    y = jnp.dot(x_ref[...] * n_out, w_ref[...],
                preferred_element_type=jnp.float32)
    dq = D_H // 4
    for q, o in enumerate((o0, o1, o2, o3)):
        o[...] = y[:, q * dq:(q + 1) * dq]


def _tc1(x, deg16, w1):
    return pl.pallas_call(
        _tc1_body,
        grid=(GRID,),
        in_specs=[
            pl.BlockSpec((BM, D_IN), lambda i: (i, 0)),
            pl.BlockSpec((1, BM, 16), lambda i: (0, i, 0)),
            pl.BlockSpec((D_IN, D_H), lambda i: (0, 0)),
        ],
        out_specs=[pl.BlockSpec((BM, D_H // 4), lambda i: (i, 0))] * 4,
        out_shape=[jax.ShapeDtypeStruct((N, D_H // 4), jnp.float32)] * 4,
    )(x, deg16, w1)


def _tc2_body(a0, a1, a2, a3, deg_ref, b1_ref, w_ref, o_ref):
    n_out = lax.rsqrt(jnp.maximum(deg_ref[0, :, 0:1], 1.0))
    n_in = lax.rsqrt(jnp.maximum(deg_ref[1, :, 0:1], 1.0))
    agg = jnp.concatenate([a0[...], a1[...], a2[...], a3[...]], axis=1)
    h = jax.nn.relu(agg * n_in + b1_ref[...]) * n_out
    o_ref[...] = jnp.dot(h, w_ref[...], preferred_element_type=jnp.float32)


def _tc2(aggs, deg16, b1, w2):
    return pl.pallas_call(
        _tc2_body,
        grid=(GRID,),
        in_specs=[pl.BlockSpec((BM, D_H // 4), lambda i: (i, 0))] * 4 + [
            pl.BlockSpec((2, BM, 16), lambda i: (0, i, 0)),
            pl.BlockSpec((1, D_H), lambda i: (0, 0)),
            pl.BlockSpec((D_H, D_OUT), lambda i: (0, 0)),
        ],
        out_specs=pl.BlockSpec((BM, D_OUT), lambda i: (i, 0)),
        out_shape=jax.ShapeDtypeStruct((N, D_OUT), jnp.float32),
    )(*aggs, deg16, b1, w2)


def _tc3_body(a_ref, deg_ref, b2_ref, o_ref):
    n_in = lax.rsqrt(jnp.maximum(deg_ref[0, :, 0:1], 1.0))
    agg = a_ref[0] + a_ref[1]
    o_ref[...] = agg * n_in + b2_ref[...]


def _tc3(a2, deg16, b2):
    return pl.pallas_call(
        _tc3_body,
        grid=(GRID,),
        in_specs=[
            pl.BlockSpec((2, BM, D_OUT), lambda i: (0, i, 0)),
            pl.BlockSpec((1, BM, 16), lambda i: (1, i, 0)),
            pl.BlockSpec((1, D_OUT), lambda i: (0, 0)),
        ],
        out_specs=pl.BlockSpec((BM, D_OUT), lambda i: (i, 0)),
        out_shape=jax.ShapeDtypeStruct((N, D_OUT), jnp.float32),
    )(a2, deg16, b2)


def kernel(in_feat, edge_index, W1, b1, W2, b2):
    e4d = edge_index.reshape(NC, NS, CCD, KD)
    srcH = edge_index[0].reshape(NS, CCH, KH)
    dstH = edge_index[1].reshape(NS, CCH, KH)
    srcO = edge_index[0].reshape(NC, NS, CCO, KO)
    dstO = edge_index[1].reshape(NC, NS, CCO, KO)
    z16 = jnp.zeros((RCH, 16), jnp.float32)
    zh = jnp.zeros((RCH, D_H // 4), jnp.float32)
    zo = jnp.zeros((RCH, D_OUT), jnp.float32)
    ones_rows = jnp.zeros((KD, 16), jnp.float32).at[:, 0].set(1.0)

    deg16 = _deg_kernel(e4d, z16, ones_rows)
    hw = _tc1(in_feat, deg16, W1)
    a1 = _agg_h(*hw, srcH, dstH, zh)
    g = _tc2(a1, deg16, b1.reshape(1, D_H), W2)
    a2 = _agg_o(g, srcO, dstO, zo)
    return _tc3(a2, deg16, b2.reshape(1, D_OUT))
